# baseline (speedup 1.0000x reference)
import functools

import jax
import jax.numpy as jnp
import numpy as np
from jax import lax
from jax.experimental import pallas as pl
from jax.experimental.pallas import tpu as pltpu
from jax.experimental.pallas import tpu_sc as plsc

S, B, H, E, TOPK = 8192, 4, 4096, 64, 8
T = S * B
BLK = 1024

NC, NS, L = 2, 16, 16
NW = NC * NS
CHUNK = T // NW
GROUPS = CHUNK // L


def _router_block(x_ref, w_ref, logits_ref, packed_ref):
    l = jnp.dot(x_ref[...], w_ref[...], preferred_element_type=jnp.float32)
    logits_ref[...] = l
    m = jnp.max(l, axis=1, keepdims=True)
    e = jnp.exp(l - m)
    a = e / jnp.sum(e, axis=1, keepdims=True)
    bits = a.view(jnp.uint32)
    sign = bits & jnp.uint32(0x80000000)
    expo = (bits >> jnp.uint32(23)) & jnp.uint32(0xFF)
    mant = bits & jnp.uint32(0x7FFFFF)
    hi = sign | ((expo + jnp.uint32(896)) << jnp.uint32(20)) | (mant >> jnp.uint32(3))
    lo = mant << jnp.uint32(29)
    zero = expo == jnp.uint32(0)
    hi = jnp.where(zero, jnp.uint32(0), hi)
    lo = jnp.where(zero, jnp.uint32(0), lo)
    packed_ref[...] = jnp.stack([lo, hi], axis=1).reshape(2 * BLK, E)


def _dense_stage(x, w):
    z = np.int32(0)
    return pl.pallas_call(
        _router_block,
        grid=(T // BLK,),
        in_specs=[
            pl.BlockSpec((BLK, H), lambda i: (i, z)),
            pl.BlockSpec((H, E), lambda i: (z, z)),
        ],
        out_specs=[
            pl.BlockSpec((BLK, E), lambda i: (i, z)),
            pl.BlockSpec((2 * BLK, E), lambda i: (i, z)),
        ],
        out_shape=[
            jax.ShapeDtypeStruct((T, E), jnp.float32),
            jax.ShapeDtypeStruct((2 * T, E), jnp.uint32),
        ],
    )(x, w)


def _topk_body(lg_hbm, out_hbm, lg_v, out_v):
    wid = lax.axis_index("s") * jnp.int32(NC) + lax.axis_index("c")
    base = wid * jnp.int32(CHUNK)
    pltpu.sync_copy(lg_hbm.at[pl.ds(base * jnp.int32(E), CHUNK * E)], lg_v)

    lanes = lax.iota(jnp.int32, L)

    def group(g, carry):
        t0 = g * jnp.int32(L)
        pos0 = (t0 + lanes) * jnp.int32(E)
        best = [jnp.full((L,), -jnp.inf, jnp.float32) for _ in range(TOPK)]
        bidx = [jnp.zeros((L,), jnp.int32) for _ in range(TOPK)]
        for e in range(E):
            cv = plsc.load_gather(lg_v, [pos0 + jnp.int32(e)])
            ci = jnp.full((L,), e, jnp.int32)
            for j in range(TOPK):
                m = cv > best[j]
                nb = jnp.where(m, cv, best[j])
                ni = jnp.where(m, ci, bidx[j])
                cv = jnp.where(m, best[j], cv)
                ci = jnp.where(m, bidx[j], ci)
                best[j] = nb
                bidx[j] = ni
        outpos = (t0 + lanes) * jnp.int32(TOPK)
        for j in range(TOPK):
            plsc.store_scatter(out_v, [outpos + jnp.int32(j)], bidx[j])
        return carry

    lax.fori_loop(jnp.int32(0), jnp.int32(GROUPS), group, None)
    pltpu.sync_copy(out_v, out_hbm.at[pl.ds(base * jnp.int32(TOPK), CHUNK * TOPK)])


@functools.cache
def _topk_stage():
    return pl.kernel(
        _topk_body,
        mesh=plsc.VectorSubcoreMesh(core_axis_name="c", subcore_axis_name="s"),
        out_type=jax.ShapeDtypeStruct((T * TOPK,), jnp.int32),
        scratch_types=[
            pltpu.VMEM((CHUNK * E,), jnp.float32),
            pltpu.VMEM((CHUNK * TOPK,), jnp.int32),
        ],
        compiler_params=pltpu.CompilerParams(needs_layout_passes=False),
        cost_estimate=pl.CostEstimate(
            flops=T * E * TOPK * 2, bytes_accessed=T * (E + TOPK) * 4,
            transcendentals=0,
        ),
    )


def kernel(hidden_states, router_weight):
    x = hidden_states.reshape(T, H)
    logits, packed = _dense_stage(x, router_weight)
    expert_index = _topk_stage()(logits.reshape(T * E)).reshape(T, TOPK)
    aff = lax.bitcast_convert_type(
        packed.reshape(T, 2, E).transpose(0, 2, 1), jnp.float64
    )
    return logits, aff, expert_index


# speedup vs baseline: 1.0500x; 1.0500x over previous
import functools

import jax
import jax.numpy as jnp
import numpy as np
from jax import lax
from jax.experimental import pallas as pl
from jax.experimental.pallas import tpu as pltpu
from jax.experimental.pallas import tpu_sc as plsc

S, B, H, E, TOPK = 8192, 4, 4096, 64, 8
T = S * B
BLK = 512

NC, NS, L = 2, 16, 16
NW = NC * NS
CHUNK = T // NW
GROUPS = CHUNK // L


def _router_block(x_ref, w_ref, logits_ref, packed_ref):
    l = jnp.dot(x_ref[...], w_ref[...], preferred_element_type=jnp.float32)
    logits_ref[...] = l
    m = jnp.max(l, axis=1, keepdims=True)
    e = jnp.exp(l - m)
    a = e / jnp.sum(e, axis=1, keepdims=True)
    bits = a.view(jnp.uint32)
    sign = bits & jnp.uint32(0x80000000)
    expo = (bits >> jnp.uint32(23)) & jnp.uint32(0xFF)
    mant = bits & jnp.uint32(0x7FFFFF)
    hi = sign | ((expo + jnp.uint32(896)) << jnp.uint32(20)) | (mant >> jnp.uint32(3))
    lo = mant << jnp.uint32(29)
    zero = expo == jnp.uint32(0)
    hi = jnp.where(zero, jnp.uint32(0), hi)
    lo = jnp.where(zero, jnp.uint32(0), lo)
    packed_ref[...] = jnp.stack([lo, hi], axis=1).reshape(2 * BLK, E)


def _dense_stage(x, w):
    z = np.int32(0)
    return pl.pallas_call(
        _router_block,
        grid=(T // BLK,),
        in_specs=[
            pl.BlockSpec((BLK, H), lambda i: (i, z)),
            pl.BlockSpec((H, E), lambda i: (z, z)),
        ],
        out_specs=[
            pl.BlockSpec((BLK, E), lambda i: (i, z)),
            pl.BlockSpec((2 * BLK, E), lambda i: (i, z)),
        ],
        out_shape=[
            jax.ShapeDtypeStruct((T, E), jnp.float32),
            jax.ShapeDtypeStruct((2 * T, E), jnp.uint32),
        ],
    )(x, w)


def _topk_body(lg_hbm, out_hbm, lg_v, out_v):
    wid = lax.axis_index("s") * jnp.int32(NC) + lax.axis_index("c")
    base = wid * jnp.int32(CHUNK)
    pltpu.sync_copy(lg_hbm.at[pl.ds(base * jnp.int32(E), CHUNK * E)], lg_v)

    lanes = lax.iota(jnp.int32, L)

    def group(g, carry):
        t0 = g * jnp.int32(L)
        pos0 = (t0 + lanes) * jnp.int32(E)
        best = [jnp.full((L,), -jnp.inf, jnp.float32) for _ in range(TOPK)]
        bidx = [jnp.zeros((L,), jnp.int32) for _ in range(TOPK)]
        for e in range(E):
            cv = plsc.load_gather(lg_v, [pos0 + jnp.int32(e)])
            ci = jnp.full((L,), e, jnp.int32)
            for j in range(TOPK):
                m = cv > best[j]
                nb = jnp.where(m, cv, best[j])
                ni = jnp.where(m, ci, bidx[j])
                cv = jnp.where(m, best[j], cv)
                ci = jnp.where(m, bidx[j], ci)
                best[j] = nb
                bidx[j] = ni
        outpos = (t0 + lanes) * jnp.int32(TOPK)
        for j in range(TOPK):
            plsc.store_scatter(out_v, [outpos + jnp.int32(j)], bidx[j])
        return carry

    lax.fori_loop(jnp.int32(0), jnp.int32(GROUPS), group, None)
    pltpu.sync_copy(out_v, out_hbm.at[pl.ds(base * jnp.int32(TOPK), CHUNK * TOPK)])


@functools.cache
def _topk_stage():
    return pl.kernel(
        _topk_body,
        mesh=plsc.VectorSubcoreMesh(core_axis_name="c", subcore_axis_name="s"),
        out_type=jax.ShapeDtypeStruct((T * TOPK,), jnp.int32),
        scratch_types=[
            pltpu.VMEM((CHUNK * E,), jnp.float32),
            pltpu.VMEM((CHUNK * TOPK,), jnp.int32),
        ],
        compiler_params=pltpu.CompilerParams(needs_layout_passes=False),
    )


def kernel(hidden_states, router_weight):
    x = hidden_states.reshape(T, H)
    logits, packed = _dense_stage(x, router_weight)
    expert_index = _topk_stage()(logits.reshape(T * E)).reshape(T, TOPK)
    aff = lax.bitcast_convert_type(
        packed.reshape(T, 2, E).transpose(0, 2, 1), jnp.float64
    )
    return logits, aff, expert_index


# speedup vs baseline: 1.0577x; 1.0073x over previous
import functools

import jax
import jax.numpy as jnp
import numpy as np
from jax import lax
from jax.experimental import pallas as pl
from jax.experimental.pallas import tpu as pltpu
from jax.experimental.pallas import tpu_sc as plsc

S, B, H, E, TOPK = 8192, 4, 4096, 64, 8
T = S * B
BLK = 1024

NC, NS, L = 2, 16, 16
NW = NC * NS
CHUNK = T // NW
GROUPS = CHUNK // L


def _router_block(x_ref, w_ref, logits_ref, packed_ref):
    l = jnp.dot(x_ref[...], w_ref[...], preferred_element_type=jnp.float32)
    logits_ref[...] = l
    m = jnp.max(l, axis=1, keepdims=True)
    e = jnp.exp(l - m)
    a = e / jnp.sum(e, axis=1, keepdims=True)
    bits = a.view(jnp.uint32)
    sign = bits & jnp.uint32(0x80000000)
    expo = (bits >> jnp.uint32(23)) & jnp.uint32(0xFF)
    mant = bits & jnp.uint32(0x7FFFFF)
    hi = sign | ((expo + jnp.uint32(896)) << jnp.uint32(20)) | (mant >> jnp.uint32(3))
    lo = mant << jnp.uint32(29)
    zero = expo == jnp.uint32(0)
    hi = jnp.where(zero, jnp.uint32(0), hi)
    lo = jnp.where(zero, jnp.uint32(0), lo)
    packed_ref[...] = jnp.stack([lo, hi], axis=1).reshape(2 * BLK, E)


def _dense_stage(x, w):
    z = np.int32(0)
    return pl.pallas_call(
        _router_block,
        grid=(T // BLK,),
        in_specs=[
            pl.BlockSpec((BLK, H), lambda i: (i, z)),
            pl.BlockSpec((H, E), lambda i: (z, z)),
        ],
        out_specs=[
            pl.BlockSpec((BLK, E), lambda i: (i, z)),
            pl.BlockSpec((2 * BLK, E), lambda i: (i, z)),
        ],
        out_shape=[
            jax.ShapeDtypeStruct((T, E), jnp.float32),
            jax.ShapeDtypeStruct((2 * T, E), jnp.uint32),
        ],
    )(x, w)


def _topk_body(lg_hbm, out_hbm, lg_v, out_v):
    wid = lax.axis_index("s") * jnp.int32(NC) + lax.axis_index("c")
    base = wid * jnp.int32(CHUNK)
    pltpu.sync_copy(lg_hbm.at[pl.ds(base * jnp.int32(E), CHUNK * E)], lg_v)

    lanes = lax.iota(jnp.int32, L)

    def group(g, carry):
        t0 = g * jnp.int32(L)
        pos0 = (t0 + lanes) * jnp.int32(E)
        best = [jnp.full((L,), jnp.int32(-2147483647), jnp.int32)
                for _ in range(TOPK)]
        for e in range(E):
            cv = plsc.load_gather(lg_v, [pos0 + jnp.int32(e)])
            b = plsc.bitcast(cv, jnp.int32)
            s = b ^ ((b >> jnp.int32(31)) & jnp.int32(0x7FFFFFFF))
            key = (s & jnp.int32(~63)) | jnp.int32(63 - e)
            for j in range(TOPK):
                nb = jnp.maximum(key, best[j])
                key = jnp.minimum(key, best[j])
                best[j] = nb
        outpos = (t0 + lanes) * jnp.int32(TOPK)
        for j in range(TOPK):
            idx = jnp.int32(63) - (best[j] & jnp.int32(63))
            plsc.store_scatter(out_v, [outpos + jnp.int32(j)], idx)
        return carry

    lax.fori_loop(jnp.int32(0), jnp.int32(GROUPS), group, None)
    pltpu.sync_copy(out_v, out_hbm.at[pl.ds(base * jnp.int32(TOPK), CHUNK * TOPK)])


@functools.cache
def _topk_stage():
    return pl.kernel(
        _topk_body,
        mesh=plsc.VectorSubcoreMesh(core_axis_name="c", subcore_axis_name="s"),
        out_type=jax.ShapeDtypeStruct((T * TOPK,), jnp.int32),
        scratch_types=[
            pltpu.VMEM((CHUNK * E,), jnp.float32),
            pltpu.VMEM((CHUNK * TOPK,), jnp.int32),
        ],
        compiler_params=pltpu.CompilerParams(needs_layout_passes=False),
    )


def kernel(hidden_states, router_weight):
    x = hidden_states.reshape(T, H)
    logits, packed = _dense_stage(x, router_weight)
    expert_index = _topk_stage()(logits.reshape(T * E)).reshape(T, TOPK)
    aff = lax.bitcast_convert_type(
        packed.reshape(T, 2, E).transpose(0, 2, 1), jnp.float64
    )
    return logits, aff, expert_index


# speedup vs baseline: 1.0660x; 1.0078x over previous
import functools

import jax
import jax.numpy as jnp
import numpy as np
from jax import lax
from jax.experimental import pallas as pl
from jax.experimental.pallas import tpu as pltpu
from jax.experimental.pallas import tpu_sc as plsc

S, B, H, E, TOPK = 8192, 4, 4096, 64, 8
T = S * B
BLK = 1024

NC, NS, L = 2, 16, 16
NW = NC * NS
CHUNK = T // NW
GROUPS = CHUNK // L


def _router_block(x_ref, w_ref, logits_ref, packed_ref):
    l = jnp.dot(x_ref[...], w_ref[...], preferred_element_type=jnp.float32)
    logits_ref[...] = l
    m = jnp.max(l, axis=1, keepdims=True)
    e = jnp.exp(l - m)
    a = e / jnp.sum(e, axis=1, keepdims=True)
    bits = a.view(jnp.uint32)
    sign = bits & jnp.uint32(0x80000000)
    expo = (bits >> jnp.uint32(23)) & jnp.uint32(0xFF)
    mant = bits & jnp.uint32(0x7FFFFF)
    hi = sign | ((expo + jnp.uint32(896)) << jnp.uint32(20)) | (mant >> jnp.uint32(3))
    lo = mant << jnp.uint32(29)
    zero = expo == jnp.uint32(0)
    hi = jnp.where(zero, jnp.uint32(0), hi)
    lo = jnp.where(zero, jnp.uint32(0), lo)
    packed_ref[...] = jnp.stack([lo, hi], axis=1).reshape(2 * BLK, E)


def _dense_stage(x, w):
    z = np.int32(0)
    return pl.pallas_call(
        _router_block,
        grid=(T // BLK,),
        in_specs=[
            pl.BlockSpec((BLK, H), lambda i: (i, z)),
            pl.BlockSpec((H, E), lambda i: (z, z)),
        ],
        out_specs=[
            pl.BlockSpec((BLK, E), lambda i: (i, z)),
            pl.BlockSpec((2 * BLK, E), lambda i: (i, z)),
        ],
        out_shape=[
            jax.ShapeDtypeStruct((T, E), jnp.float32),
            jax.ShapeDtypeStruct((2 * T, E), jnp.uint32),
        ],
    )(x, w)


HALF = CHUNK // 2


def _topk_body(lg_hbm, out_hbm, lg_v, out_v):
    wid = lax.axis_index("s") * jnp.int32(NC) + lax.axis_index("c")
    base = wid * jnp.int32(CHUNK)
    lanes = lax.iota(jnp.int32, L)

    def half(h, carry):
        hbase = base + h * jnp.int32(HALF)
        pltpu.sync_copy(lg_hbm.at[pl.ds(hbase, HALF), :], lg_v)

        def group(g, carry2):
            tok_l = g * jnp.int32(L) + lanes
            tok = tok_l + h * jnp.int32(HALF)
            best = [jnp.full((L,), jnp.int32(-2147483647), jnp.int32)
                    for _ in range(TOPK)]
            for e in range(E):
                cv = plsc.load_gather(lg_v, [tok_l, jnp.full((L,), e, jnp.int32)])
                b = plsc.bitcast(cv, jnp.int32)
                s = b ^ ((b >> jnp.int32(31)) & jnp.int32(0x7FFFFFFF))
                key = (s & jnp.int32(~63)) | jnp.int32(63 - e)
                for j in range(TOPK):
                    nb = jnp.maximum(key, best[j])
                    key = jnp.minimum(key, best[j])
                    best[j] = nb
            outpos = tok * jnp.int32(TOPK)
            for j in range(TOPK):
                idx = jnp.int32(63) - (best[j] & jnp.int32(63))
                plsc.store_scatter(out_v, [outpos + jnp.int32(j)], idx)
            return carry2

        lax.fori_loop(jnp.int32(0), jnp.int32(HALF // L), group, None)
        return carry

    lax.fori_loop(jnp.int32(0), jnp.int32(2), half, None)
    pltpu.sync_copy(out_v, out_hbm.at[pl.ds(base * jnp.int32(TOPK), CHUNK * TOPK)])


@functools.cache
def _topk_stage():
    return pl.kernel(
        _topk_body,
        mesh=plsc.VectorSubcoreMesh(core_axis_name="c", subcore_axis_name="s"),
        out_type=jax.ShapeDtypeStruct((T * TOPK,), jnp.int32),
        scratch_types=[
            pltpu.VMEM((HALF, E), jnp.float32),
            pltpu.VMEM((CHUNK * TOPK,), jnp.int32),
        ],
        compiler_params=pltpu.CompilerParams(needs_layout_passes=False),
    )


def kernel(hidden_states, router_weight):
    x = hidden_states.reshape(T, H)
    logits, packed = _dense_stage(x, router_weight)
    expert_index = _topk_stage()(logits).reshape(T, TOPK)
    aff = lax.bitcast_convert_type(
        packed.reshape(T, 2, E).transpose(0, 2, 1), jnp.float64
    )
    return logits, aff, expert_index


# speedup vs baseline: 1.0668x; 1.0007x over previous
import functools

import jax
import jax.numpy as jnp
import numpy as np
from jax import lax
from jax.experimental import pallas as pl
from jax.experimental.pallas import tpu as pltpu
from jax.experimental.pallas import tpu_sc as plsc

S, B, H, E, TOPK = 8192, 4, 4096, 64, 8
T = S * B
BLK = 1024

NC, NS, L = 2, 16, 16
NW = NC * NS
CHUNK = T // NW
HALF = CHUNK // 2


def _router_block(x_ref, w_ref, logits_ref, packed_ref):
    l = jnp.dot(x_ref[...], w_ref[...], preferred_element_type=jnp.float32)
    logits_ref[...] = l
    m = jnp.max(l, axis=1, keepdims=True)
    e = jnp.exp(l - m)
    a = e / jnp.sum(e, axis=1, keepdims=True)
    bits = a.view(jnp.uint32)
    sign = bits & jnp.uint32(0x80000000)
    expo = (bits >> jnp.uint32(23)) & jnp.uint32(0xFF)
    mant = bits & jnp.uint32(0x7FFFFF)
    hi = sign | ((expo + jnp.uint32(896)) << jnp.uint32(20)) | (mant >> jnp.uint32(3))
    lo = mant << jnp.uint32(29)
    zero = expo == jnp.uint32(0)
    hi = jnp.where(zero, jnp.uint32(0), hi)
    lo = jnp.where(zero, jnp.uint32(0), lo)
    packed_ref[...] = jnp.stack([lo, hi], axis=1).reshape(2 * BLK, E)


def _dense_stage(x, w):
    z = np.int32(0)
    return pl.pallas_call(
        _router_block,
        grid=(T // BLK,),
        in_specs=[
            pl.BlockSpec((BLK, H), lambda i: (i, z)),
            pl.BlockSpec((H, E), lambda i: (z, z)),
        ],
        out_specs=[
            pl.BlockSpec((BLK, E), lambda i: (i, z)),
            pl.BlockSpec((2 * BLK, E), lambda i: (i, z)),
        ],
        out_shape=[
            jax.ShapeDtypeStruct((T, E), jnp.float32),
            jax.ShapeDtypeStruct((2 * T, E), jnp.uint32),
        ],
    )(x, w)


def _topk_body(lg_hbm, out_hbm, lg_v, out_v):
    wid = lax.axis_index("s") * jnp.int32(NC) + lax.axis_index("c")
    base = wid * jnp.int32(CHUNK)
    lanes = lax.iota(jnp.int32, L)

    def half(h, carry):
        hbase = base + h * jnp.int32(HALF)
        pltpu.sync_copy(lg_hbm.at[pl.ds(hbase, HALF), :], lg_v)

        def group(g, carry2):
            tok_l = g * jnp.int32(L) + lanes
            tok = tok_l + h * jnp.int32(HALF)
            best = [jnp.full((L,), -jnp.inf, jnp.float32) for _ in range(TOPK)]
            bidx = [jnp.zeros((L,), jnp.int32) for _ in range(TOPK)]
            for e in range(E):
                cv = plsc.load_gather(lg_v, [tok_l, jnp.full((L,), e, jnp.int32)])
                ci = jnp.full((L,), e, jnp.int32)
                for j in range(TOPK):
                    m = cv > best[j]
                    nb = jnp.maximum(cv, best[j])
                    cv = jnp.minimum(cv, best[j])
                    ni = jnp.where(m, ci, bidx[j])
                    ci = jnp.where(m, bidx[j], ci)
                    best[j] = nb
                    bidx[j] = ni
            outpos = tok * jnp.int32(TOPK)
            for j in range(TOPK):
                plsc.store_scatter(out_v, [outpos + jnp.int32(j)], bidx[j])
            return carry2

        lax.fori_loop(jnp.int32(0), jnp.int32(HALF // L), group, None)
        return carry

    lax.fori_loop(jnp.int32(0), jnp.int32(2), half, None)
    pltpu.sync_copy(out_v, out_hbm.at[pl.ds(base * jnp.int32(TOPK), CHUNK * TOPK)])


@functools.cache
def _topk_stage():
    return pl.kernel(
        _topk_body,
        mesh=plsc.VectorSubcoreMesh(core_axis_name="c", subcore_axis_name="s"),
        out_type=jax.ShapeDtypeStruct((T * TOPK,), jnp.int32),
        scratch_types=[
            pltpu.VMEM((HALF, E), jnp.float32),
            pltpu.VMEM((CHUNK * TOPK,), jnp.int32),
        ],
        compiler_params=pltpu.CompilerParams(needs_layout_passes=False),
    )


def kernel(hidden_states, router_weight):
    x = hidden_states.reshape(T, H)
    logits, packed = _dense_stage(x, router_weight)
    expert_index = _topk_stage()(logits).reshape(T, TOPK)
    aff = lax.bitcast_convert_type(
        packed.reshape(T, 2, E).transpose(0, 2, 1), jnp.float64
    )
    return logits, aff, expert_index
